# Initial kernel scaffold; baseline (speedup 1.0000x reference)
import jax
import jax.numpy as jnp
from jax.experimental import pallas as pl


def kernel(x, edge_index, W_in, b_in, W_res, b_res, convW0, convb0, bng0, bnb0, convW1, convb1, bng1, bnb1, convW2, convb2, bng2, bnb2, cW1, cb1, cW2, cb2, cW3, cb3):
    raise NotImplementedError("write your pallas kernel here")


# speedup vs baseline: 11.1148x; 11.1148x over previous
import functools

import jax
import jax.numpy as jnp
from jax import lax
from jax.experimental import pallas as pl
from jax.experimental.pallas import tpu as pltpu
from jax.experimental.pallas import tpu_sc as plsc

_N = 10000
_E = 320000
_H = 128
_EPS = 1e-5
_NC = 2
_NS = 16
_NW = _NC * _NS
_EPW = _E // _NW
_K = 80
_CHUNKS = _EPW // _K
_NPAD = 10240
_RPS = _NPAD // _NS
_DEGW = 16


def _sc_mesh():
    return plsc.VectorSubcoreMesh(core_axis_name="c", subcore_axis_name="s")


def _sc_degree(dst, zeros):
    @functools.partial(
        pl.kernel,
        out_type=jax.ShapeDtypeStruct((_NC * _NPAD,), jnp.float32),
        mesh=_sc_mesh(),
        scratch_types=[
            pltpu.VMEM((_K,), jnp.int32),
            pltpu.VMEM((_K,), jnp.float32),
            pltpu.VMEM_SHARED((_NPAD,), jnp.float32),
        ],
    )
    def deg_kernel(dst_hbm, zeros_hbm, out_hbm, dst_v, ones_v, acc_sh):
        c = lax.axis_index("c")
        s = lax.axis_index("s")
        wid = c * _NS + s
        row0 = s * _RPS
        pltpu.sync_copy(zeros_hbm.at[pl.ds(row0, _RPS)], acc_sh.at[pl.ds(row0, _RPS)])

        @pl.loop(0, _K, step=16)
        def _(j):
            ones_v[pl.ds(j, 16)] = jnp.ones((16,), jnp.float32)

        plsc.subcore_barrier()

        @pl.loop(0, _CHUNKS)
        def _(i):
            base = wid * _EPW + i * _K
            pltpu.sync_copy(dst_hbm.at[pl.ds(base, _K)], dst_v)
            pltpu.sync_copy(ones_v, acc_sh.at[dst_v], add=True)

        plsc.subcore_barrier()
        pltpu.sync_copy(acc_sh.at[pl.ds(row0, _RPS)],
                        out_hbm.at[pl.ds(c * _NPAD + row0, _RPS)])

    return deg_kernel(dst, zeros)


def _sc_aggregate(y, src, dst, zeros):
    @functools.partial(
        pl.kernel,
        out_type=jax.ShapeDtypeStruct((_NC, _NPAD, _H), jnp.float32),
        mesh=_sc_mesh(),
        scratch_types=[
            pltpu.VMEM((_K,), jnp.int32),
            pltpu.VMEM((_K,), jnp.int32),
            pltpu.VMEM((_K, _H), jnp.float32),
            pltpu.SemaphoreType.DMA,
            pltpu.VMEM_SHARED((_NPAD, _H), jnp.float32),
        ],
    )
    def agg_kernel(y_hbm, src_hbm, dst_hbm, zeros_hbm, out_hbm,
                   src_v, dst_v, rows_v, sem, acc_sh):
        c = lax.axis_index("c")
        s = lax.axis_index("s")
        wid = c * _NS + s
        row0 = s * _RPS
        pltpu.sync_copy(zeros_hbm.at[pl.ds(row0, _RPS)], acc_sh.at[pl.ds(row0, _RPS)])
        plsc.subcore_barrier()

        @pl.loop(0, _CHUNKS)
        def _(i):
            base = wid * _EPW + i * _K
            pltpu.sync_copy(src_hbm.at[pl.ds(base, _K)], src_v)
            pltpu.sync_copy(dst_hbm.at[pl.ds(base, _K)], dst_v)
            pltpu.async_copy(y_hbm.at[src_v], rows_v, sem).wait()
            pltpu.sync_copy(rows_v, acc_sh.at[dst_v], add=True)

        plsc.subcore_barrier()
        pltpu.sync_copy(acc_sh.at[pl.ds(row0, _RPS)], out_hbm.at[c, pl.ds(row0, _RPS)])

    return agg_kernel(y, src, dst, zeros)


def _bn_relu(agg, g, b):
    mu = jnp.mean(agg, axis=0, keepdims=True)
    d = agg - mu
    var = jnp.mean(d * d, axis=0, keepdims=True)
    return jnp.maximum(d * lax.rsqrt(var + _EPS) * g + b, 0.0)


def _dot(a, b):
    return jnp.dot(a, b, preferred_element_type=jnp.float32)


def _tc_prep(x, w_in, b_in, w_res, b_res, w0, deg0, deg1):
    def body(x_ref, wi_ref, bi_ref, wr_ref, br_ref, w0_ref, d0_ref, d1_ref,
             dis_ref, res_ref, y0_ref):
        deg = d0_ref[...] + d1_ref[...] + 1.0
        dis = lax.rsqrt(deg)
        dis_ref[...] = dis
        x = x_ref[...]
        xp = _dot(x, wi_ref[...]) + bi_ref[...]
        res_ref[...] = _dot(x, wr_ref[...]) + br_ref[...]
        y0_ref[...] = _dot(xp, w0_ref[...]) * dis

    return pl.pallas_call(
        body,
        out_shape=(
            jax.ShapeDtypeStruct((_N, 1), jnp.float32),
            jax.ShapeDtypeStruct((_N, _H), jnp.float32),
            jax.ShapeDtypeStruct((_N, _H), jnp.float32),
        ),
    )(x, w_in, b_in, w_res, b_res, w0, deg0, deg1)


def _tc_layer0(p, y, dis, g, b, w_next):
    def body(p_ref, y_ref, dis_ref, g_ref, b_ref, w_ref, y1_ref):
        dis = dis_ref[...]
        agg = (p_ref[0, : _N, :] + p_ref[1, : _N, :] + y_ref[...]) * dis
        xn = _bn_relu(agg, g_ref[...], b_ref[...])
        y1_ref[...] = _dot(xn, w_ref[...]) * dis

    return pl.pallas_call(
        body,
        out_shape=jax.ShapeDtypeStruct((_N, _H), jnp.float32),
    )(p, y, dis, g, b, w_next)


def _tc_layer1(p, y, dis, g, b, res, w_next):
    def body(p_ref, y_ref, dis_ref, g_ref, b_ref, res_ref, w_ref, y2_ref):
        dis = dis_ref[...]
        agg = (p_ref[0, : _N, :] + p_ref[1, : _N, :] + y_ref[...]) * dis
        xn = _bn_relu(agg, g_ref[...], b_ref[...])
        xp = xn + res_ref[...]
        y2_ref[...] = _dot(xp, w_ref[...]) * dis

    return pl.pallas_call(
        body,
        out_shape=jax.ShapeDtypeStruct((_N, _H), jnp.float32),
    )(p, y, dis, g, b, res, w_next)


def _tc_layer2_head(p, y, dis, g, b, cw1, cb1, cw2, cb2, cw3, cb3):
    def body(p_ref, y_ref, dis_ref, g_ref, b_ref, w1_ref, b1_ref, w2_ref,
             b2_ref, w3_ref, b3_ref, out_ref):
        dis = dis_ref[...]
        agg = (p_ref[0, : _N, :] + p_ref[1, : _N, :] + y_ref[...]) * dis
        xn = _bn_relu(agg, g_ref[...], b_ref[...])
        h1 = jnp.maximum(_dot(xn, w1_ref[...]) + b1_ref[...], 0.0)
        h2 = jnp.maximum(_dot(h1, w2_ref[...]) + b2_ref[...], 0.0)
        out_ref[...] = _dot(h2, w3_ref[...]) + b3_ref[...]

    return pl.pallas_call(
        body,
        out_shape=jax.ShapeDtypeStruct((_N, 2), jnp.float32),
    )(p, y, dis, g, b, cw1, cb1, cw2, cb2, cw3, cb3)


def kernel(x, edge_index, W_in, b_in, W_res, b_res,
           convW0, convb0, bng0, bnb0,
           convW1, convb1, bng1, bnb1,
           convW2, convb2, bng2, bnb2,
           cW1, cb1, cW2, cb2, cW3, cb3):
    src = edge_index[0]
    dst = edge_index[1]
    zeros_deg = jnp.zeros((_NPAD,), jnp.float32)
    zeros_acc = jnp.zeros((_NPAD, _H), jnp.float32)

    degf = _sc_degree(dst, zeros_deg)
    deg0 = degf[:_N].reshape(_N, 1)
    deg1 = degf[_NPAD:_NPAD + _N].reshape(_N, 1)
    dis, res, y0 = _tc_prep(x, W_in, b_in.reshape(1, -1), W_res,
                            b_res.reshape(1, -1), convW0, deg0, deg1)
    p0 = _sc_aggregate(y0, src, dst, zeros_acc)
    y1 = _tc_layer0(p0, y0, dis, bng0.reshape(1, -1), bnb0.reshape(1, -1), convW1)
    p1 = _sc_aggregate(y1, src, dst, zeros_acc)
    y2 = _tc_layer1(p1, y1, dis, bng1.reshape(1, -1), bnb1.reshape(1, -1),
                    res, convW2)
    p2 = _sc_aggregate(y2, src, dst, zeros_acc)
    out = _tc_layer2_head(p2, y2, dis, bng2.reshape(1, -1), bnb2.reshape(1, -1),
                          cW1, cb1.reshape(1, -1), cW2, cb2.reshape(1, -1),
                          cW3, cb3.reshape(1, -1))
    return out


# speedup vs baseline: 16.6307x; 1.4963x over previous
import functools

import jax
import jax.numpy as jnp
from jax import lax
from jax.experimental import pallas as pl
from jax.experimental.pallas import tpu as pltpu
from jax.experimental.pallas import tpu_sc as plsc

_N = 10000
_E = 320000
_H = 128
_EPS = 1e-5
_NC = 2
_NS = 16
_NW = _NC * _NS
_EPW = _E // _NW
_K = 200
_CHUNKS = _EPW // _K
_NPAD = 10240
_RPS = _NPAD // _NS
_DEGW = 16


def _sc_mesh():
    return plsc.VectorSubcoreMesh(core_axis_name="c", subcore_axis_name="s")


def _sc_degree(dst, zeros):
    @functools.partial(
        pl.kernel,
        out_type=jax.ShapeDtypeStruct((_NC * _NPAD,), jnp.float32),
        mesh=_sc_mesh(),
        scratch_types=[
            pltpu.VMEM((_K,), jnp.int32),
            pltpu.VMEM((_K,), jnp.float32),
            pltpu.VMEM_SHARED((_NPAD,), jnp.float32),
        ],
    )
    def deg_kernel(dst_hbm, zeros_hbm, out_hbm, dst_v, ones_v, acc_sh):
        c = lax.axis_index("c")
        s = lax.axis_index("s")
        wid = c * _NS + s
        row0 = s * _RPS
        pltpu.sync_copy(zeros_hbm.at[pl.ds(row0, _RPS)], acc_sh.at[pl.ds(row0, _RPS)])

        @pl.loop(0, _K, step=16)
        def _(j):
            ones_v[pl.ds(j, 16)] = jnp.ones((16,), jnp.float32)

        plsc.subcore_barrier()

        @pl.loop(0, _CHUNKS)
        def _(i):
            base = wid * _EPW + i * _K
            pltpu.sync_copy(dst_hbm.at[pl.ds(base, _K)], dst_v)
            pltpu.sync_copy(ones_v, acc_sh.at[dst_v], add=True)

        plsc.subcore_barrier()
        pltpu.sync_copy(acc_sh.at[pl.ds(row0, _RPS)],
                        out_hbm.at[pl.ds(c * _NPAD + row0, _RPS)])

    return deg_kernel(dst, zeros)


def _sc_aggregate(y, src, dst, zeros):
    @functools.partial(
        pl.kernel,
        out_type=jax.ShapeDtypeStruct((_NC, _NPAD, _H), jnp.float32),
        mesh=_sc_mesh(),
        scratch_types=[
            pltpu.VMEM((_K,), jnp.int32),
            pltpu.VMEM((_K,), jnp.int32),
            pltpu.VMEM((_K, _H), jnp.float32),
            pltpu.SemaphoreType.DMA,
            pltpu.VMEM_SHARED((_NPAD, _H), jnp.float32),
        ],
    )
    def agg_kernel(y_hbm, src_hbm, dst_hbm, zeros_hbm, out_hbm,
                   src_v, dst_v, rows_v, sem, acc_sh):
        c = lax.axis_index("c")
        s = lax.axis_index("s")
        wid = c * _NS + s
        row0 = s * _RPS
        pltpu.sync_copy(zeros_hbm.at[pl.ds(row0, _RPS)], acc_sh.at[pl.ds(row0, _RPS)])
        plsc.subcore_barrier()

        @pl.loop(0, _CHUNKS)
        def _(i):
            base = wid * _EPW + i * _K
            pltpu.sync_copy(src_hbm.at[pl.ds(base, _K)], src_v)
            pltpu.sync_copy(dst_hbm.at[pl.ds(base, _K)], dst_v)
            pltpu.async_copy(y_hbm.at[src_v], rows_v, sem).wait()
            pltpu.sync_copy(rows_v, acc_sh.at[dst_v], add=True)

        plsc.subcore_barrier()
        pltpu.sync_copy(acc_sh.at[pl.ds(row0, _RPS)], out_hbm.at[c, pl.ds(row0, _RPS)])

    return agg_kernel(y, src, dst, zeros)


def _bn_relu(agg, g, b):
    mu = jnp.mean(agg, axis=0, keepdims=True)
    d = agg - mu
    var = jnp.mean(d * d, axis=0, keepdims=True)
    return jnp.maximum(d * lax.rsqrt(var + _EPS) * g + b, 0.0)


def _dot(a, b):
    return jnp.dot(a, b, preferred_element_type=jnp.float32)


def _tc_prep(x, w_in, b_in, w_res, b_res, w0, deg0, deg1):
    def body(x_ref, wi_ref, bi_ref, wr_ref, br_ref, w0_ref, d0_ref, d1_ref,
             dis_ref, res_ref, y0_ref):
        deg = d0_ref[...] + d1_ref[...] + 1.0
        dis = lax.rsqrt(deg)
        dis_ref[...] = dis
        x = x_ref[...]
        xp = _dot(x, wi_ref[...]) + bi_ref[...]
        res_ref[...] = _dot(x, wr_ref[...]) + br_ref[...]
        y0_ref[...] = _dot(xp, w0_ref[...]) * dis

    return pl.pallas_call(
        body,
        out_shape=(
            jax.ShapeDtypeStruct((_N, 1), jnp.float32),
            jax.ShapeDtypeStruct((_N, _H), jnp.float32),
            jax.ShapeDtypeStruct((_N, _H), jnp.float32),
        ),
    )(x, w_in, b_in, w_res, b_res, w0, deg0, deg1)


def _tc_layer0(p, y, dis, g, b, w_next):
    def body(p_ref, y_ref, dis_ref, g_ref, b_ref, w_ref, y1_ref):
        dis = dis_ref[...]
        agg = (p_ref[0, : _N, :] + p_ref[1, : _N, :] + y_ref[...]) * dis
        xn = _bn_relu(agg, g_ref[...], b_ref[...])
        y1_ref[...] = _dot(xn, w_ref[...]) * dis

    return pl.pallas_call(
        body,
        out_shape=jax.ShapeDtypeStruct((_N, _H), jnp.float32),
    )(p, y, dis, g, b, w_next)


def _tc_layer1(p, y, dis, g, b, res, w_next):
    def body(p_ref, y_ref, dis_ref, g_ref, b_ref, res_ref, w_ref, y2_ref):
        dis = dis_ref[...]
        agg = (p_ref[0, : _N, :] + p_ref[1, : _N, :] + y_ref[...]) * dis
        xn = _bn_relu(agg, g_ref[...], b_ref[...])
        xp = xn + res_ref[...]
        y2_ref[...] = _dot(xp, w_ref[...]) * dis

    return pl.pallas_call(
        body,
        out_shape=jax.ShapeDtypeStruct((_N, _H), jnp.float32),
    )(p, y, dis, g, b, res, w_next)


def _tc_layer2_head(p, y, dis, g, b, cw1, cb1, cw2, cb2, cw3, cb3):
    def body(p_ref, y_ref, dis_ref, g_ref, b_ref, w1_ref, b1_ref, w2_ref,
             b2_ref, w3_ref, b3_ref, out_ref):
        dis = dis_ref[...]
        agg = (p_ref[0, : _N, :] + p_ref[1, : _N, :] + y_ref[...]) * dis
        xn = _bn_relu(agg, g_ref[...], b_ref[...])
        h1 = jnp.maximum(_dot(xn, w1_ref[...]) + b1_ref[...], 0.0)
        h2 = jnp.maximum(_dot(h1, w2_ref[...]) + b2_ref[...], 0.0)
        out_ref[...] = _dot(h2, w3_ref[...]) + b3_ref[...]

    return pl.pallas_call(
        body,
        out_shape=jax.ShapeDtypeStruct((_N, 2), jnp.float32),
    )(p, y, dis, g, b, cw1, cb1, cw2, cb2, cw3, cb3)


def kernel(x, edge_index, W_in, b_in, W_res, b_res,
           convW0, convb0, bng0, bnb0,
           convW1, convb1, bng1, bnb1,
           convW2, convb2, bng2, bnb2,
           cW1, cb1, cW2, cb2, cW3, cb3):
    src = edge_index[0]
    dst = edge_index[1]
    zeros_deg = jnp.zeros((_NPAD,), jnp.float32)
    zeros_acc = jnp.zeros((_NPAD, _H), jnp.float32)

    degf = _sc_degree(dst, zeros_deg)
    deg0 = degf[:_N].reshape(_N, 1)
    deg1 = degf[_NPAD:_NPAD + _N].reshape(_N, 1)
    dis, res, y0 = _tc_prep(x, W_in, b_in.reshape(1, -1), W_res,
                            b_res.reshape(1, -1), convW0, deg0, deg1)
    p0 = _sc_aggregate(y0, src, dst, zeros_acc)
    y1 = _tc_layer0(p0, y0, dis, bng0.reshape(1, -1), bnb0.reshape(1, -1), convW1)
    p1 = _sc_aggregate(y1, src, dst, zeros_acc)
    y2 = _tc_layer1(p1, y1, dis, bng1.reshape(1, -1), bnb1.reshape(1, -1),
                    res, convW2)
    p2 = _sc_aggregate(y2, src, dst, zeros_acc)
    out = _tc_layer2_head(p2, y2, dis, bng2.reshape(1, -1), bnb2.reshape(1, -1),
                          cW1, cb1.reshape(1, -1), cW2, cb2.reshape(1, -1),
                          cW3, cb3.reshape(1, -1))
    return out


# speedup vs baseline: 21.4749x; 1.2913x over previous
import functools

import jax
import jax.numpy as jnp
from jax import lax
from jax.experimental import pallas as pl
from jax.experimental.pallas import tpu as pltpu
from jax.experimental.pallas import tpu_sc as plsc

_N = 10000
_E = 320000
_H = 128
_EPS = 1e-5
_NC = 2
_NS = 16
_NW = _NC * _NS
_K = 144
_EP = 322560
_EPW = _EP // _NW
_CHUNKS = _EPW // _K
_NPAD = 10240
_RPS = _NPAD // _NS


def _sc_mesh():
    return plsc.VectorSubcoreMesh(core_axis_name="c", subcore_axis_name="s")


def _sc_degree(dst, zeros):
    @functools.partial(
        pl.kernel,
        out_type=jax.ShapeDtypeStruct((_NC * _NPAD,), jnp.float32),
        mesh=_sc_mesh(),
        scratch_types=[
            pltpu.VMEM((_K,), jnp.int32),
            pltpu.VMEM((_K,), jnp.float32),
            pltpu.VMEM_SHARED((_NPAD,), jnp.float32),
        ],
    )
    def deg_kernel(dst_hbm, zeros_hbm, out_hbm, dst_v, ones_v, acc_sh):
        c = lax.axis_index("c")
        s = lax.axis_index("s")
        wid = c * _NS + s
        row0 = s * _RPS
        pltpu.sync_copy(zeros_hbm.at[pl.ds(row0, _RPS)], acc_sh.at[pl.ds(row0, _RPS)])

        @pl.loop(0, _K, step=16)
        def _(j):
            ones_v[pl.ds(j, 16)] = jnp.ones((16,), jnp.float32)

        plsc.subcore_barrier()

        @pl.loop(0, _CHUNKS)
        def _(i):
            base = wid * _EPW + i * _K
            pltpu.sync_copy(dst_hbm.at[pl.ds(base, _K)], dst_v)
            pltpu.sync_copy(ones_v, acc_sh.at[dst_v], add=True)

        plsc.subcore_barrier()
        pltpu.sync_copy(acc_sh.at[pl.ds(row0, _RPS)],
                        out_hbm.at[pl.ds(c * _NPAD + row0, _RPS)])

    return deg_kernel(dst, zeros)


def _sc_aggregate(y, src, dst, zeros):
    @functools.partial(
        pl.kernel,
        out_type=jax.ShapeDtypeStruct((_NC, _NPAD, _H), jnp.float32),
        mesh=_sc_mesh(),
        scratch_types=[
            pltpu.VMEM((_K,), jnp.int32),
            pltpu.VMEM((_K,), jnp.int32),
            pltpu.VMEM((_K,), jnp.int32),
            pltpu.VMEM((_K,), jnp.int32),
            pltpu.VMEM((_K, _H), jnp.float32),
            pltpu.VMEM((_K, _H), jnp.float32),
            pltpu.SemaphoreType.DMA,
            pltpu.SemaphoreType.DMA,
            pltpu.VMEM_SHARED((_NPAD, _H), jnp.float32),
        ],
    )
    def agg_kernel(y_hbm, src_hbm, dst_hbm, zeros_hbm, out_hbm,
                   src_a, dst_a, src_b, dst_b, rows_a, rows_b,
                   sem_a, sem_b, acc_sh):
        c = lax.axis_index("c")
        s = lax.axis_index("s")
        wid = c * _NS + s
        wbase = wid * _EPW
        row0 = s * _RPS
        pltpu.sync_copy(zeros_hbm.at[pl.ds(row0, _RPS)], acc_sh.at[pl.ds(row0, _RPS)])
        plsc.subcore_barrier()

        def fire(j, src_v, dst_v, rows_v, sem):
            base = wbase + j * _K
            pltpu.sync_copy(src_hbm.at[pl.ds(base, _K)], src_v)
            pltpu.sync_copy(dst_hbm.at[pl.ds(base, _K)], dst_v)
            pltpu.async_copy(y_hbm.at[src_v], rows_v, sem)

        def drain(src_v, dst_v, rows_v, sem):
            pltpu.make_async_copy(y_hbm.at[src_v], rows_v, sem).wait()
            pltpu.sync_copy(rows_v, acc_sh.at[dst_v], add=True)

        fire(0, src_a, dst_a, rows_a, sem_a)

        @pl.loop(0, _CHUNKS // 2 - 1)
        def _(t):
            fire(2 * t + 1, src_b, dst_b, rows_b, sem_b)
            drain(src_a, dst_a, rows_a, sem_a)
            fire(2 * t + 2, src_a, dst_a, rows_a, sem_a)
            drain(src_b, dst_b, rows_b, sem_b)

        fire(_CHUNKS - 1, src_b, dst_b, rows_b, sem_b)
        drain(src_a, dst_a, rows_a, sem_a)
        drain(src_b, dst_b, rows_b, sem_b)

        plsc.subcore_barrier()
        pltpu.sync_copy(acc_sh.at[pl.ds(row0, _RPS)], out_hbm.at[c, pl.ds(row0, _RPS)])

    return agg_kernel(y, src, dst, zeros)


def _bn_relu(agg, g, b):
    mu = jnp.mean(agg, axis=0, keepdims=True)
    d = agg - mu
    var = jnp.mean(d * d, axis=0, keepdims=True)
    return jnp.maximum(d * lax.rsqrt(var + _EPS) * g + b, 0.0)


def _dot(a, b):
    return jnp.dot(a, b, preferred_element_type=jnp.float32)


def _tc_proj(x, w_in, b_in, w_res, b_res, w0):
    def body(x_ref, wi_ref, bi_ref, wr_ref, br_ref, w0_ref, res_ref, xw0_ref):
        x = x_ref[...]
        xp = _dot(x, wi_ref[...]) + bi_ref[...]
        res_ref[...] = _dot(x, wr_ref[...]) + br_ref[...]
        xw0_ref[...] = _dot(xp, w0_ref[...])

    return pl.pallas_call(
        body,
        out_shape=(
            jax.ShapeDtypeStruct((_N, _H), jnp.float32),
            jax.ShapeDtypeStruct((_N, _H), jnp.float32),
        ),
    )(x, w_in, b_in, w_res, b_res, w0)


def _tc_scale(xw0, deg0, deg1):
    def body(xw0_ref, d0_ref, d1_ref, dis_ref, y0_ref):
        deg = d0_ref[...] + d1_ref[...] + 1.0
        dis = lax.rsqrt(deg)
        dis_ref[...] = dis
        y0_ref[...] = xw0_ref[...] * dis

    return pl.pallas_call(
        body,
        out_shape=(
            jax.ShapeDtypeStruct((_N, 1), jnp.float32),
            jax.ShapeDtypeStruct((_N, _H), jnp.float32),
        ),
    )(xw0, deg0, deg1)


def _tc_layer0(p, y, dis, g, b, w_next):
    def body(p_ref, y_ref, dis_ref, g_ref, b_ref, w_ref, y1_ref):
        dis = dis_ref[...]
        agg = (p_ref[0, : _N, :] + p_ref[1, : _N, :] + y_ref[...]) * dis
        xn = _bn_relu(agg, g_ref[...], b_ref[...])
        y1_ref[...] = _dot(xn, w_ref[...]) * dis

    return pl.pallas_call(
        body,
        out_shape=jax.ShapeDtypeStruct((_N, _H), jnp.float32),
    )(p, y, dis, g, b, w_next)


def _tc_layer1(p, y, dis, g, b, res, w_next):
    def body(p_ref, y_ref, dis_ref, g_ref, b_ref, res_ref, w_ref, y2_ref):
        dis = dis_ref[...]
        agg = (p_ref[0, : _N, :] + p_ref[1, : _N, :] + y_ref[...]) * dis
        xn = _bn_relu(agg, g_ref[...], b_ref[...])
        xp = xn + res_ref[...]
        y2_ref[...] = _dot(xp, w_ref[...]) * dis

    return pl.pallas_call(
        body,
        out_shape=jax.ShapeDtypeStruct((_N, _H), jnp.float32),
    )(p, y, dis, g, b, res, w_next)


def _tc_layer2_head(p, y, dis, g, b, cw1, cb1, cw2, cb2, cw3, cb3):
    def body(p_ref, y_ref, dis_ref, g_ref, b_ref, w1_ref, b1_ref, w2_ref,
             b2_ref, w3_ref, b3_ref, out_ref):
        dis = dis_ref[...]
        agg = (p_ref[0, : _N, :] + p_ref[1, : _N, :] + y_ref[...]) * dis
        xn = _bn_relu(agg, g_ref[...], b_ref[...])
        h1 = jnp.maximum(_dot(xn, w1_ref[...]) + b1_ref[...], 0.0)
        h2 = jnp.maximum(_dot(h1, w2_ref[...]) + b2_ref[...], 0.0)
        out_ref[...] = _dot(h2, w3_ref[...]) + b3_ref[...]

    return pl.pallas_call(
        body,
        out_shape=jax.ShapeDtypeStruct((_N, 2), jnp.float32),
    )(p, y, dis, g, b, cw1, cb1, cw2, cb2, cw3, cb3)


def kernel(x, edge_index, W_in, b_in, W_res, b_res,
           convW0, convb0, bng0, bnb0,
           convW1, convb1, bng1, bnb1,
           convW2, convb2, bng2, bnb2,
           cW1, cb1, cW2, cb2, cW3, cb3):
    pad = _EP - _E
    pad_i = jnp.arange(pad, dtype=jnp.int32)
    src = jnp.concatenate([edge_index[0], (pad_i * 977) % _N])
    dst = jnp.concatenate([edge_index[1], _N + pad_i % (_NPAD - _N)])
    zeros_deg = jnp.zeros((_NPAD,), jnp.float32)
    zeros_acc = jnp.zeros((_NPAD, _H), jnp.float32)

    degf = _sc_degree(dst, zeros_deg)
    deg0 = degf[:_N].reshape(_N, 1)
    deg1 = degf[_NPAD:_NPAD + _N].reshape(_N, 1)
    res, xw0 = _tc_proj(x, W_in, b_in.reshape(1, -1), W_res,
                        b_res.reshape(1, -1), convW0)
    dis, y0 = _tc_scale(xw0, deg0, deg1)
    p0 = _sc_aggregate(y0, src, dst, zeros_acc)
    y1 = _tc_layer0(p0, y0, dis, bng0.reshape(1, -1), bnb0.reshape(1, -1), convW1)
    p1 = _sc_aggregate(y1, src, dst, zeros_acc)
    y2 = _tc_layer1(p1, y1, dis, bng1.reshape(1, -1), bnb1.reshape(1, -1),
                    res, convW2)
    p2 = _sc_aggregate(y2, src, dst, zeros_acc)
    out = _tc_layer2_head(p2, y2, dis, bng2.reshape(1, -1), bnb2.reshape(1, -1),
                          cW1, cb1.reshape(1, -1), cW2, cb2.reshape(1, -1),
                          cW3, cb3.reshape(1, -1))
    return out


# speedup vs baseline: 25.7160x; 1.1975x over previous
import functools

import jax
import jax.numpy as jnp
from jax import lax
from jax.experimental import pallas as pl
from jax.experimental.pallas import tpu as pltpu
from jax.experimental.pallas import tpu_sc as plsc

_N = 10000
_E = 320000
_H = 128
_EPS = 1e-5
_NC = 2
_NS = 16
_NW = _NC * _NS
_K = 128
_EP = 327680
_EPW = _EP // _NW
_CHUNKS = _EPW // _K
_BLK = 16
_NBLK = _CHUNKS // _BLK
_NPAD = 10240
_RPS = _NPAD // _NS


def _sc_mesh():
    return plsc.VectorSubcoreMesh(core_axis_name="c", subcore_axis_name="s")


def _sc_degree(dst, zeros):
    @functools.partial(
        pl.kernel,
        out_type=jax.ShapeDtypeStruct((_NC * _NPAD,), jnp.float32),
        mesh=_sc_mesh(),
        scratch_types=[
            pltpu.VMEM((_CHUNKS, _K), jnp.int32),
            pltpu.VMEM((_K,), jnp.float32),
            pltpu.VMEM_SHARED((_NPAD,), jnp.float32),
        ],
    )
    def deg_kernel(dst_hbm, zeros_hbm, out_hbm, dst_v, ones_v, acc_sh):
        c = lax.axis_index("c")
        s = lax.axis_index("s")
        wid = c * _NS + s
        row0 = s * _RPS
        pltpu.sync_copy(zeros_hbm.at[pl.ds(row0, _RPS)], acc_sh.at[pl.ds(row0, _RPS)])
        pltpu.sync_copy(dst_hbm.at[pl.ds(wid * _CHUNKS, _CHUNKS)], dst_v)

        @pl.loop(0, _K, step=16)
        def _(j):
            ones_v[pl.ds(j, 16)] = jnp.ones((16,), jnp.float32)

        plsc.subcore_barrier()

        @pl.loop(0, _CHUNKS)
        def _(i):
            pltpu.sync_copy(ones_v, acc_sh.at[dst_v.at[i]], add=True)

        plsc.subcore_barrier()
        pltpu.sync_copy(acc_sh.at[pl.ds(row0, _RPS)],
                        out_hbm.at[pl.ds(c * _NPAD + row0, _RPS)])

    return deg_kernel(dst, zeros)


def _sc_aggregate(y, src, dst, zeros):
    @functools.partial(
        pl.kernel,
        out_type=jax.ShapeDtypeStruct((_NC, _NPAD, _H), jnp.float32),
        mesh=_sc_mesh(),
        scratch_types=[
            pltpu.VMEM((_BLK, _K), jnp.int32),
            pltpu.VMEM((_BLK, _K), jnp.int32),
            pltpu.VMEM((_K, _H), jnp.float32),
            pltpu.VMEM((_K, _H), jnp.float32),
            pltpu.SemaphoreType.DMA,
            pltpu.SemaphoreType.DMA,
            pltpu.VMEM_SHARED((_NPAD, _H), jnp.float32),
        ],
    )
    def agg_kernel(y_hbm, src_hbm, dst_hbm, zeros_hbm, out_hbm,
                   src_blk, dst_blk, rows_a, rows_b, sem_a, sem_b, acc_sh):
        c = lax.axis_index("c")
        s = lax.axis_index("s")
        wid = c * _NS + s
        wrow = wid * _CHUNKS
        row0 = s * _RPS
        pltpu.sync_copy(zeros_hbm.at[pl.ds(row0, _RPS)], acc_sh.at[pl.ds(row0, _RPS)])
        plsc.subcore_barrier()

        def fire(r, rows_v, sem):
            pltpu.async_copy(y_hbm.at[src_blk.at[r]], rows_v, sem)

        def drain(r, rows_v, sem):
            pltpu.make_async_copy(y_hbm.at[src_blk.at[r]], rows_v, sem).wait()
            pltpu.sync_copy(rows_v, acc_sh.at[dst_blk.at[r]], add=True)

        @pl.loop(0, _NBLK)
        def _(b):
            pltpu.sync_copy(src_hbm.at[pl.ds(wrow + b * _BLK, _BLK)], src_blk)
            pltpu.sync_copy(dst_hbm.at[pl.ds(wrow + b * _BLK, _BLK)], dst_blk)
            fire(0, rows_a, sem_a)

            @pl.loop(0, _BLK // 2 - 1)
            def _(t):
                fire(2 * t + 1, rows_b, sem_b)
                drain(2 * t, rows_a, sem_a)
                fire(2 * t + 2, rows_a, sem_a)
                drain(2 * t + 1, rows_b, sem_b)

            fire(_BLK - 1, rows_b, sem_b)
            drain(_BLK - 2, rows_a, sem_a)
            drain(_BLK - 1, rows_b, sem_b)

        plsc.subcore_barrier()
        pltpu.sync_copy(acc_sh.at[pl.ds(row0, _RPS)], out_hbm.at[c, pl.ds(row0, _RPS)])

    return agg_kernel(y, src, dst, zeros)


def _bn_relu(agg, g, b):
    mu = jnp.mean(agg, axis=0, keepdims=True)
    d = agg - mu
    var = jnp.mean(d * d, axis=0, keepdims=True)
    return jnp.maximum(d * lax.rsqrt(var + _EPS) * g + b, 0.0)


def _dot(a, b):
    return jnp.dot(a, b, preferred_element_type=jnp.float32)


def _tc_proj(x, w_in, b_in, w_res, b_res, w0):
    def body(x_ref, wi_ref, bi_ref, wr_ref, br_ref, w0_ref, res_ref, xw0_ref):
        x = x_ref[...]
        xp = _dot(x, wi_ref[...]) + bi_ref[...]
        res_ref[...] = _dot(x, wr_ref[...]) + br_ref[...]
        xw0_ref[...] = _dot(xp, w0_ref[...])

    return pl.pallas_call(
        body,
        out_shape=(
            jax.ShapeDtypeStruct((_N, _H), jnp.float32),
            jax.ShapeDtypeStruct((_N, _H), jnp.float32),
        ),
    )(x, w_in, b_in, w_res, b_res, w0)


def _tc_scale(xw0, deg0, deg1):
    def body(xw0_ref, d0_ref, d1_ref, dis_ref, y0_ref):
        deg = d0_ref[...] + d1_ref[...] + 1.0
        dis = lax.rsqrt(deg)
        dis_ref[...] = dis
        y0_ref[...] = xw0_ref[...] * dis

    return pl.pallas_call(
        body,
        out_shape=(
            jax.ShapeDtypeStruct((_N, 1), jnp.float32),
            jax.ShapeDtypeStruct((_N, _H), jnp.float32),
        ),
    )(xw0, deg0, deg1)


def _tc_layer0(p, y, dis, g, b, w_next):
    def body(p_ref, y_ref, dis_ref, g_ref, b_ref, w_ref, y1_ref):
        dis = dis_ref[...]
        agg = (p_ref[0, : _N, :] + p_ref[1, : _N, :] + y_ref[...]) * dis
        xn = _bn_relu(agg, g_ref[...], b_ref[...])
        y1_ref[...] = _dot(xn, w_ref[...]) * dis

    return pl.pallas_call(
        body,
        out_shape=jax.ShapeDtypeStruct((_N, _H), jnp.float32),
    )(p, y, dis, g, b, w_next)


def _tc_layer1(p, y, dis, g, b, res, w_next):
    def body(p_ref, y_ref, dis_ref, g_ref, b_ref, res_ref, w_ref, y2_ref):
        dis = dis_ref[...]
        agg = (p_ref[0, : _N, :] + p_ref[1, : _N, :] + y_ref[...]) * dis
        xn = _bn_relu(agg, g_ref[...], b_ref[...])
        xp = xn + res_ref[...]
        y2_ref[...] = _dot(xp, w_ref[...]) * dis

    return pl.pallas_call(
        body,
        out_shape=jax.ShapeDtypeStruct((_N, _H), jnp.float32),
    )(p, y, dis, g, b, res, w_next)


def _tc_layer2_head(p, y, dis, g, b, cw1, cb1, cw2, cb2, cw3, cb3):
    def body(p_ref, y_ref, dis_ref, g_ref, b_ref, w1_ref, b1_ref, w2_ref,
             b2_ref, w3_ref, b3_ref, out_ref):
        dis = dis_ref[...]
        agg = (p_ref[0, : _N, :] + p_ref[1, : _N, :] + y_ref[...]) * dis
        xn = _bn_relu(agg, g_ref[...], b_ref[...])
        h1 = jnp.maximum(_dot(xn, w1_ref[...]) + b1_ref[...], 0.0)
        h2 = jnp.maximum(_dot(h1, w2_ref[...]) + b2_ref[...], 0.0)
        out_ref[...] = _dot(h2, w3_ref[...]) + b3_ref[...]

    return pl.pallas_call(
        body,
        out_shape=jax.ShapeDtypeStruct((_N, 2), jnp.float32),
    )(p, y, dis, g, b, cw1, cb1, cw2, cb2, cw3, cb3)


def kernel(x, edge_index, W_in, b_in, W_res, b_res,
           convW0, convb0, bng0, bnb0,
           convW1, convb1, bng1, bnb1,
           convW2, convb2, bng2, bnb2,
           cW1, cb1, cW2, cb2, cW3, cb3):
    pad = _EP - _E
    pad_i = jnp.arange(pad, dtype=jnp.int32)
    src = jnp.concatenate([edge_index[0], (pad_i * 977) % _N]).reshape(-1, _K)
    dst = jnp.concatenate([edge_index[1], _N + pad_i % (_NPAD - _N)]).reshape(-1, _K)
    zeros_deg = jnp.zeros((_NPAD,), jnp.float32)
    zeros_acc = jnp.zeros((_NPAD, _H), jnp.float32)

    degf = _sc_degree(dst, zeros_deg)
    deg0 = degf[:_N].reshape(_N, 1)
    deg1 = degf[_NPAD:_NPAD + _N].reshape(_N, 1)
    res, xw0 = _tc_proj(x, W_in, b_in.reshape(1, -1), W_res,
                        b_res.reshape(1, -1), convW0)
    dis, y0 = _tc_scale(xw0, deg0, deg1)
    p0 = _sc_aggregate(y0, src, dst, zeros_acc)
    y1 = _tc_layer0(p0, y0, dis, bng0.reshape(1, -1), bnb0.reshape(1, -1), convW1)
    p1 = _sc_aggregate(y1, src, dst, zeros_acc)
    y2 = _tc_layer1(p1, y1, dis, bng1.reshape(1, -1), bnb1.reshape(1, -1),
                    res, convW2)
    p2 = _sc_aggregate(y2, src, dst, zeros_acc)
    out = _tc_layer2_head(p2, y2, dis, bng2.reshape(1, -1), bnb2.reshape(1, -1),
                          cW1, cb1.reshape(1, -1), cW2, cb2.reshape(1, -1),
                          cW3, cb3.reshape(1, -1))
    return out


# speedup vs baseline: 27.5879x; 1.0728x over previous
import functools

import jax
import jax.numpy as jnp
from jax import lax
from jax.experimental import pallas as pl
from jax.experimental.pallas import tpu as pltpu
from jax.experimental.pallas import tpu_sc as plsc

_N = 10000
_E = 320000
_H = 128
_EPS = 1e-5
_NC = 2
_NS = 16
_NW = _NC * _NS
_K = 128
_EP = 327680
_EPW = _EP // _NW
_CHUNKS = _EPW // _K
_BLK = 16
_NBLK = _CHUNKS // _BLK
_NPAD = 10240
_RPS = _NPAD // _NS


def _sc_mesh():
    return plsc.VectorSubcoreMesh(core_axis_name="c", subcore_axis_name="s")


def _sc_degree(dst, zeros):
    @functools.partial(
        pl.kernel,
        out_type=jax.ShapeDtypeStruct((_NC * _NPAD,), jnp.float32),
        mesh=_sc_mesh(),
        scratch_types=[
            pltpu.VMEM((_CHUNKS, _K), jnp.int32),
            pltpu.VMEM((_K,), jnp.float32),
            pltpu.VMEM_SHARED((_NPAD,), jnp.float32),
        ],
    )
    def deg_kernel(dst_hbm, zeros_hbm, out_hbm, dst_v, ones_v, acc_sh):
        c = lax.axis_index("c")
        s = lax.axis_index("s")
        wid = c * _NS + s
        row0 = s * _RPS
        pltpu.sync_copy(zeros_hbm.at[pl.ds(row0, _RPS)], acc_sh.at[pl.ds(row0, _RPS)])
        pltpu.sync_copy(dst_hbm.at[pl.ds(wid * _CHUNKS, _CHUNKS)], dst_v)

        @pl.loop(0, _K, step=16)
        def _(j):
            ones_v[pl.ds(j, 16)] = jnp.ones((16,), jnp.float32)

        plsc.subcore_barrier()

        @pl.loop(0, _CHUNKS)
        def _(i):
            pltpu.sync_copy(ones_v, acc_sh.at[dst_v.at[i]], add=True)

        plsc.subcore_barrier()
        pltpu.sync_copy(acc_sh.at[pl.ds(row0, _RPS)],
                        out_hbm.at[pl.ds(c * _NPAD + row0, _RPS)])

    return deg_kernel(dst, zeros)


def _sc_aggregate(y, src, dst, zeros):
    @functools.partial(
        pl.kernel,
        out_type=jax.ShapeDtypeStruct((_NC, _NPAD, _H), jnp.float32),
        mesh=_sc_mesh(),
        scratch_types=[
            pltpu.VMEM((_BLK, _K), jnp.int32),
            pltpu.VMEM((_BLK, _K), jnp.int32),
            pltpu.VMEM((_BLK, _K), jnp.int32),
            pltpu.VMEM((_BLK, _K), jnp.int32),
            pltpu.VMEM((_K, _H), jnp.float32),
            pltpu.VMEM((_K, _H), jnp.float32),
            pltpu.SemaphoreType.DMA,
            pltpu.SemaphoreType.DMA,
            pltpu.SemaphoreType.DMA,
            pltpu.SemaphoreType.DMA,
            pltpu.VMEM_SHARED((_NPAD, _H), jnp.float32),
        ],
    )
    def agg_kernel(y_hbm, src_hbm, dst_hbm, zeros_hbm, out_hbm,
                   src_x, dst_x, src_y, dst_y, rows_a, rows_b,
                   sem_a, sem_b, sem_ix, sem_iy, acc_sh):
        c = lax.axis_index("c")
        s = lax.axis_index("s")
        wid = c * _NS + s
        wrow = wid * _CHUNKS
        row0 = s * _RPS

        def load_idx(b, sbuf, dbuf, sem):
            pltpu.async_copy(src_hbm.at[pl.ds(wrow + b * _BLK, _BLK)], sbuf, sem)
            pltpu.async_copy(dst_hbm.at[pl.ds(wrow + b * _BLK, _BLK)], dbuf, sem)

        def wait_idx(b, sbuf, dbuf, sem):
            pltpu.make_async_copy(src_hbm.at[pl.ds(wrow + b * _BLK, _BLK)], sbuf, sem).wait()
            pltpu.make_async_copy(dst_hbm.at[pl.ds(wrow + b * _BLK, _BLK)], dbuf, sem).wait()

        def fire(sbuf, r, rows_v, sem):
            pltpu.async_copy(y_hbm.at[sbuf.at[r]], rows_v, sem)

        def drain(sbuf, dbuf, r, rows_v, sem):
            pltpu.make_async_copy(y_hbm.at[sbuf.at[r]], rows_v, sem).wait()
            pltpu.sync_copy(rows_v, acc_sh.at[dbuf.at[r]], add=True)

        def block(sbuf, dbuf, cross_fire):
            @pl.loop(0, _BLK // 2 - 1)
            def _(t):
                fire(sbuf, 2 * t + 1, rows_b, sem_b)
                drain(sbuf, dbuf, 2 * t, rows_a, sem_a)
                fire(sbuf, 2 * t + 2, rows_a, sem_a)
                drain(sbuf, dbuf, 2 * t + 1, rows_b, sem_b)

            fire(sbuf, _BLK - 1, rows_b, sem_b)
            drain(sbuf, dbuf, _BLK - 2, rows_a, sem_a)
            cross_fire()
            drain(sbuf, dbuf, _BLK - 1, rows_b, sem_b)

        load_idx(0, src_x, dst_x, sem_ix)
        load_idx(1, src_y, dst_y, sem_iy)
        pltpu.sync_copy(zeros_hbm.at[pl.ds(row0, _RPS)], acc_sh.at[pl.ds(row0, _RPS)])
        plsc.subcore_barrier()

        wait_idx(0, src_x, dst_x, sem_ix)
        fire(src_x, 0, rows_a, sem_a)

        def cross01():
            wait_idx(1, src_y, dst_y, sem_iy)
            fire(src_y, 0, rows_a, sem_a)

        block(src_x, dst_x, cross01)
        load_idx(2, src_x, dst_x, sem_ix)

        def cross12():
            wait_idx(2, src_x, dst_x, sem_ix)
            fire(src_x, 0, rows_a, sem_a)

        block(src_y, dst_y, cross12)
        load_idx(3, src_y, dst_y, sem_iy)

        def cross23():
            wait_idx(3, src_y, dst_y, sem_iy)
            fire(src_y, 0, rows_a, sem_a)

        block(src_x, dst_x, cross23)
        load_idx(4, src_x, dst_x, sem_ix)

        def cross34():
            wait_idx(4, src_x, dst_x, sem_ix)
            fire(src_x, 0, rows_a, sem_a)

        block(src_y, dst_y, cross34)
        block(src_x, dst_x, lambda: None)

        plsc.subcore_barrier()
        pltpu.sync_copy(acc_sh.at[pl.ds(row0, _RPS)], out_hbm.at[c, pl.ds(row0, _RPS)])

    return agg_kernel(y, src, dst, zeros)


def _bn_relu(agg, g, b):
    mu = jnp.mean(agg, axis=0, keepdims=True)
    d = agg - mu
    var = jnp.mean(d * d, axis=0, keepdims=True)
    return jnp.maximum(d * lax.rsqrt(var + _EPS) * g + b, 0.0)


def _dot(a, b):
    return jnp.dot(a, b, preferred_element_type=jnp.float32)


def _tc_proj(x, w_in, b_in, w_res, b_res, w0):
    def body(x_ref, wi_ref, bi_ref, wr_ref, br_ref, w0_ref, res_ref, xw0_ref):
        x = x_ref[...]
        xp = _dot(x, wi_ref[...]) + bi_ref[...]
        res_ref[...] = _dot(x, wr_ref[...]) + br_ref[...]
        xw0_ref[...] = _dot(xp, w0_ref[...])

    return pl.pallas_call(
        body,
        out_shape=(
            jax.ShapeDtypeStruct((_N, _H), jnp.float32),
            jax.ShapeDtypeStruct((_N, _H), jnp.float32),
        ),
    )(x, w_in, b_in, w_res, b_res, w0)


def _tc_scale(xw0, deg0, deg1):
    def body(xw0_ref, d0_ref, d1_ref, dis_ref, y0_ref):
        deg = d0_ref[...] + d1_ref[...] + 1.0
        dis = lax.rsqrt(deg)
        dis_ref[...] = dis
        y0_ref[...] = xw0_ref[...] * dis

    return pl.pallas_call(
        body,
        out_shape=(
            jax.ShapeDtypeStruct((_N, 1), jnp.float32),
            jax.ShapeDtypeStruct((_N, _H), jnp.float32),
        ),
    )(xw0, deg0, deg1)


def _tc_layer0(p, y, dis, g, b, w_next):
    def body(p_ref, y_ref, dis_ref, g_ref, b_ref, w_ref, y1_ref):
        dis = dis_ref[...]
        agg = (p_ref[0, : _N, :] + p_ref[1, : _N, :] + y_ref[...]) * dis
        xn = _bn_relu(agg, g_ref[...], b_ref[...])
        y1_ref[...] = _dot(xn, w_ref[...]) * dis

    return pl.pallas_call(
        body,
        out_shape=jax.ShapeDtypeStruct((_N, _H), jnp.float32),
    )(p, y, dis, g, b, w_next)


def _tc_layer1(p, y, dis, g, b, res, w_next):
    def body(p_ref, y_ref, dis_ref, g_ref, b_ref, res_ref, w_ref, y2_ref):
        dis = dis_ref[...]
        agg = (p_ref[0, : _N, :] + p_ref[1, : _N, :] + y_ref[...]) * dis
        xn = _bn_relu(agg, g_ref[...], b_ref[...])
        xp = xn + res_ref[...]
        y2_ref[...] = _dot(xp, w_ref[...]) * dis

    return pl.pallas_call(
        body,
        out_shape=jax.ShapeDtypeStruct((_N, _H), jnp.float32),
    )(p, y, dis, g, b, res, w_next)


def _tc_layer2_head(p, y, dis, g, b, cw1, cb1, cw2, cb2, cw3, cb3):
    def body(p_ref, y_ref, dis_ref, g_ref, b_ref, w1_ref, b1_ref, w2_ref,
             b2_ref, w3_ref, b3_ref, out_ref):
        dis = dis_ref[...]
        agg = (p_ref[0, : _N, :] + p_ref[1, : _N, :] + y_ref[...]) * dis
        xn = _bn_relu(agg, g_ref[...], b_ref[...])
        h1 = jnp.maximum(_dot(xn, w1_ref[...]) + b1_ref[...], 0.0)
        h2 = jnp.maximum(_dot(h1, w2_ref[...]) + b2_ref[...], 0.0)
        out_ref[...] = _dot(h2, w3_ref[...]) + b3_ref[...]

    return pl.pallas_call(
        body,
        out_shape=jax.ShapeDtypeStruct((_N, 2), jnp.float32),
    )(p, y, dis, g, b, cw1, cb1, cw2, cb2, cw3, cb3)


def kernel(x, edge_index, W_in, b_in, W_res, b_res,
           convW0, convb0, bng0, bnb0,
           convW1, convb1, bng1, bnb1,
           convW2, convb2, bng2, bnb2,
           cW1, cb1, cW2, cb2, cW3, cb3):
    pad = _EP - _E
    pad_i = jnp.arange(pad, dtype=jnp.int32)
    src = jnp.concatenate([edge_index[0], (pad_i * 977) % _N]).reshape(-1, _K)
    dst = jnp.concatenate([edge_index[1], _N + pad_i % (_NPAD - _N)]).reshape(-1, _K)
    zeros_deg = jnp.zeros((_NPAD,), jnp.float32)
    zeros_acc = jnp.zeros((_NPAD, _H), jnp.float32)

    degf = _sc_degree(dst, zeros_deg)
    deg0 = degf[:_N].reshape(_N, 1)
    deg1 = degf[_NPAD:_NPAD + _N].reshape(_N, 1)
    res, xw0 = _tc_proj(x, W_in, b_in.reshape(1, -1), W_res,
                        b_res.reshape(1, -1), convW0)
    dis, y0 = _tc_scale(xw0, deg0, deg1)
    p0 = _sc_aggregate(y0, src, dst, zeros_acc)
    y1 = _tc_layer0(p0, y0, dis, bng0.reshape(1, -1), bnb0.reshape(1, -1), convW1)
    p1 = _sc_aggregate(y1, src, dst, zeros_acc)
    y2 = _tc_layer1(p1, y1, dis, bng1.reshape(1, -1), bnb1.reshape(1, -1),
                    res, convW2)
    p2 = _sc_aggregate(y2, src, dst, zeros_acc)
    out = _tc_layer2_head(p2, y2, dis, bng2.reshape(1, -1), bnb2.reshape(1, -1),
                          cW1, cb1.reshape(1, -1), cW2, cb2.reshape(1, -1),
                          cW3, cb3.reshape(1, -1))
    return out


# speedup vs baseline: 27.8427x; 1.0092x over previous
import functools

import jax
import jax.numpy as jnp
from jax import lax
from jax.experimental import pallas as pl
from jax.experimental.pallas import tpu as pltpu
from jax.experimental.pallas import tpu_sc as plsc

_N = 10000
_E = 320000
_H = 128
_EPS = 1e-5
_NC = 2
_NS = 16
_NW = _NC * _NS
_K = 128
_EP = 327680
_EPW = _EP // _NW
_CHUNKS = _EPW // _K
_BLK = 16
_NBLK = _CHUNKS // _BLK
_NPAD = 10240
_RPS = _NPAD // _NS


def _sc_mesh():
    return plsc.VectorSubcoreMesh(core_axis_name="c", subcore_axis_name="s")


def _sc_degree(dst, zeros):
    @functools.partial(
        pl.kernel,
        out_type=jax.ShapeDtypeStruct((_NC * _NPAD,), jnp.float32),
        mesh=_sc_mesh(),
        scratch_types=[
            pltpu.VMEM((_CHUNKS, _K), jnp.int32),
            pltpu.VMEM((_K,), jnp.float32),
            pltpu.VMEM_SHARED((_NPAD,), jnp.float32),
        ],
    )
    def deg_kernel(dst_hbm, zeros_hbm, out_hbm, dst_v, ones_v, acc_sh):
        c = lax.axis_index("c")
        s = lax.axis_index("s")
        wid = c * _NS + s
        row0 = s * _RPS
        pltpu.sync_copy(zeros_hbm.at[pl.ds(row0, _RPS)], acc_sh.at[pl.ds(row0, _RPS)])
        pltpu.sync_copy(dst_hbm.at[pl.ds(wid * _CHUNKS, _CHUNKS)], dst_v)

        @pl.loop(0, _K, step=16)
        def _(j):
            ones_v[pl.ds(j, 16)] = jnp.ones((16,), jnp.float32)

        plsc.subcore_barrier()

        @pl.loop(0, _CHUNKS)
        def _(i):
            pltpu.sync_copy(ones_v, acc_sh.at[dst_v.at[i]], add=True)

        plsc.subcore_barrier()
        pltpu.sync_copy(acc_sh.at[pl.ds(row0, _RPS)],
                        out_hbm.at[pl.ds(c * _NPAD + row0, _RPS)])

    return deg_kernel(dst, zeros)


def _sc_aggregate(y, src, dst, zeros):
    @functools.partial(
        pl.kernel,
        out_type=jax.ShapeDtypeStruct((_NC, _NPAD, _H), jnp.float32),
        mesh=_sc_mesh(),
        scratch_types=[
            pltpu.VMEM((_BLK, _K), jnp.int32),
            pltpu.VMEM((_BLK, _K), jnp.int32),
            pltpu.VMEM((_BLK, _K), jnp.int32),
            pltpu.VMEM((_BLK, _K), jnp.int32),
            pltpu.VMEM((_K, _H), jnp.float32),
            pltpu.VMEM((_K, _H), jnp.float32),
            pltpu.SemaphoreType.DMA,
            pltpu.SemaphoreType.DMA,
            pltpu.SemaphoreType.DMA,
            pltpu.SemaphoreType.DMA,
            pltpu.SemaphoreType.DMA,
            pltpu.VMEM_SHARED((_NPAD, _H), jnp.float32),
        ],
    )
    def agg_kernel(y_hbm, src_hbm, dst_hbm, zeros_hbm, out_hbm,
                   src_x, dst_x, src_y, dst_y, rows_a, rows_b,
                   sem_a, sem_b, sem_ix, sem_iy, sem_z, acc_sh):
        c = lax.axis_index("c")
        s = lax.axis_index("s")
        wid = c * _NS + s
        wrow = wid * _CHUNKS
        row0 = s * _RPS

        def load_idx(b, sbuf, dbuf, sem):
            pltpu.async_copy(src_hbm.at[pl.ds(wrow + b * _BLK, _BLK)], sbuf, sem)
            pltpu.async_copy(dst_hbm.at[pl.ds(wrow + b * _BLK, _BLK)], dbuf, sem)

        def wait_idx(b, sbuf, dbuf, sem):
            pltpu.make_async_copy(src_hbm.at[pl.ds(wrow + b * _BLK, _BLK)], sbuf, sem).wait()
            pltpu.make_async_copy(dst_hbm.at[pl.ds(wrow + b * _BLK, _BLK)], dbuf, sem).wait()

        def fire(sbuf, r, rows_v, sem):
            pltpu.async_copy(y_hbm.at[sbuf.at[r]], rows_v, sem)

        def drain(sbuf, dbuf, r, rows_v, sem):
            pltpu.make_async_copy(y_hbm.at[sbuf.at[r]], rows_v, sem).wait()
            pltpu.sync_copy(rows_v, acc_sh.at[dbuf.at[r]], add=True)

        def block(sbuf, dbuf, cross_fire):
            @pl.loop(0, _BLK // 2 - 1)
            def _(t):
                fire(sbuf, 2 * t + 1, rows_b, sem_b)
                drain(sbuf, dbuf, 2 * t, rows_a, sem_a)
                fire(sbuf, 2 * t + 2, rows_a, sem_a)
                drain(sbuf, dbuf, 2 * t + 1, rows_b, sem_b)

            fire(sbuf, _BLK - 1, rows_b, sem_b)
            drain(sbuf, dbuf, _BLK - 2, rows_a, sem_a)
            cross_fire()
            drain(sbuf, dbuf, _BLK - 1, rows_b, sem_b)

        load_idx(0, src_x, dst_x, sem_ix)
        load_idx(1, src_y, dst_y, sem_iy)
        pltpu.async_copy(zeros_hbm.at[pl.ds(row0, _RPS)],
                         acc_sh.at[pl.ds(row0, _RPS)], sem_z)
        wait_idx(0, src_x, dst_x, sem_ix)
        fire(src_x, 0, rows_a, sem_a)
        pltpu.make_async_copy(zeros_hbm.at[pl.ds(row0, _RPS)],
                              acc_sh.at[pl.ds(row0, _RPS)], sem_z).wait()
        plsc.subcore_barrier()

        def cross01():
            wait_idx(1, src_y, dst_y, sem_iy)
            fire(src_y, 0, rows_a, sem_a)

        block(src_x, dst_x, cross01)
        load_idx(2, src_x, dst_x, sem_ix)

        def cross12():
            wait_idx(2, src_x, dst_x, sem_ix)
            fire(src_x, 0, rows_a, sem_a)

        block(src_y, dst_y, cross12)
        load_idx(3, src_y, dst_y, sem_iy)

        def cross23():
            wait_idx(3, src_y, dst_y, sem_iy)
            fire(src_y, 0, rows_a, sem_a)

        block(src_x, dst_x, cross23)
        load_idx(4, src_x, dst_x, sem_ix)

        def cross34():
            wait_idx(4, src_x, dst_x, sem_ix)
            fire(src_x, 0, rows_a, sem_a)

        block(src_y, dst_y, cross34)
        block(src_x, dst_x, lambda: None)

        plsc.subcore_barrier()
        pltpu.sync_copy(acc_sh.at[pl.ds(row0, _RPS)], out_hbm.at[c, pl.ds(row0, _RPS)])

    return agg_kernel(y, src, dst, zeros)


def _bn_relu(agg, g, b):
    mu = jnp.mean(agg, axis=0, keepdims=True)
    d = agg - mu
    var = jnp.mean(d * d, axis=0, keepdims=True)
    return jnp.maximum(d * lax.rsqrt(var + _EPS) * g + b, 0.0)


def _dot(a, b):
    return jnp.dot(a, b, preferred_element_type=jnp.float32)


def _tc_proj(x, w_in, b_in, w_res, b_res, w0):
    def body(x_ref, wi_ref, bi_ref, wr_ref, br_ref, w0_ref, res_ref, xw0_ref):
        x = x_ref[...]
        xp = _dot(x, wi_ref[...]) + bi_ref[...]
        res_ref[...] = _dot(x, wr_ref[...]) + br_ref[...]
        xw0_ref[...] = _dot(xp, w0_ref[...])

    return pl.pallas_call(
        body,
        out_shape=(
            jax.ShapeDtypeStruct((_N, _H), jnp.float32),
            jax.ShapeDtypeStruct((_N, _H), jnp.float32),
        ),
    )(x, w_in, b_in, w_res, b_res, w0)


def _tc_scale(xw0, deg0, deg1):
    def body(xw0_ref, d0_ref, d1_ref, dis_ref, y0_ref):
        deg = d0_ref[...] + d1_ref[...] + 1.0
        dis = lax.rsqrt(deg)
        dis_ref[...] = dis
        y0_ref[...] = xw0_ref[...] * dis

    return pl.pallas_call(
        body,
        out_shape=(
            jax.ShapeDtypeStruct((_N, 1), jnp.float32),
            jax.ShapeDtypeStruct((_N, _H), jnp.float32),
        ),
    )(xw0, deg0, deg1)


def _tc_layer0(p, y, dis, g, b, w_next):
    def body(p_ref, y_ref, dis_ref, g_ref, b_ref, w_ref, y1_ref):
        dis = dis_ref[...]
        agg = (p_ref[0, : _N, :] + p_ref[1, : _N, :] + y_ref[...]) * dis
        xn = _bn_relu(agg, g_ref[...], b_ref[...])
        y1_ref[...] = _dot(xn, w_ref[...]) * dis

    return pl.pallas_call(
        body,
        out_shape=jax.ShapeDtypeStruct((_N, _H), jnp.float32),
    )(p, y, dis, g, b, w_next)


def _tc_layer1(p, y, dis, g, b, res, w_next):
    def body(p_ref, y_ref, dis_ref, g_ref, b_ref, res_ref, w_ref, y2_ref):
        dis = dis_ref[...]
        agg = (p_ref[0, : _N, :] + p_ref[1, : _N, :] + y_ref[...]) * dis
        xn = _bn_relu(agg, g_ref[...], b_ref[...])
        xp = xn + res_ref[...]
        y2_ref[...] = _dot(xp, w_ref[...]) * dis

    return pl.pallas_call(
        body,
        out_shape=jax.ShapeDtypeStruct((_N, _H), jnp.float32),
    )(p, y, dis, g, b, res, w_next)


def _tc_layer2_head(p, y, dis, g, b, cw1, cb1, cw2, cb2, cw3, cb3):
    def body(p_ref, y_ref, dis_ref, g_ref, b_ref, w1_ref, b1_ref, w2_ref,
             b2_ref, w3_ref, b3_ref, out_ref):
        dis = dis_ref[...]
        agg = (p_ref[0, : _N, :] + p_ref[1, : _N, :] + y_ref[...]) * dis
        xn = _bn_relu(agg, g_ref[...], b_ref[...])
        h1 = jnp.maximum(_dot(xn, w1_ref[...]) + b1_ref[...], 0.0)
        h2 = jnp.maximum(_dot(h1, w2_ref[...]) + b2_ref[...], 0.0)
        out_ref[...] = _dot(h2, w3_ref[...]) + b3_ref[...]

    return pl.pallas_call(
        body,
        out_shape=jax.ShapeDtypeStruct((_N, 2), jnp.float32),
    )(p, y, dis, g, b, cw1, cb1, cw2, cb2, cw3, cb3)


def kernel(x, edge_index, W_in, b_in, W_res, b_res,
           convW0, convb0, bng0, bnb0,
           convW1, convb1, bng1, bnb1,
           convW2, convb2, bng2, bnb2,
           cW1, cb1, cW2, cb2, cW3, cb3):
    pad = _EP - _E
    pad_i = jnp.arange(pad, dtype=jnp.int32)
    src = jnp.concatenate([edge_index[0], (pad_i * 977) % _N]).reshape(-1, _K)
    dst = jnp.concatenate([edge_index[1], _N + pad_i % (_NPAD - _N)]).reshape(-1, _K)
    zeros_deg = jnp.zeros((_NPAD,), jnp.float32)
    zeros_acc = jnp.zeros((_NPAD, _H), jnp.float32)

    degf = _sc_degree(dst, zeros_deg)
    deg0 = degf[:_N].reshape(_N, 1)
    deg1 = degf[_NPAD:_NPAD + _N].reshape(_N, 1)
    res, xw0 = _tc_proj(x, W_in, b_in.reshape(1, -1), W_res,
                        b_res.reshape(1, -1), convW0)
    dis, y0 = _tc_scale(xw0, deg0, deg1)
    p0 = _sc_aggregate(y0, src, dst, zeros_acc)
    y1 = _tc_layer0(p0, y0, dis, bng0.reshape(1, -1), bnb0.reshape(1, -1), convW1)
    p1 = _sc_aggregate(y1, src, dst, zeros_acc)
    y2 = _tc_layer1(p1, y1, dis, bng1.reshape(1, -1), bnb1.reshape(1, -1),
                    res, convW2)
    p2 = _sc_aggregate(y2, src, dst, zeros_acc)
    out = _tc_layer2_head(p2, y2, dis, bng2.reshape(1, -1), bnb2.reshape(1, -1),
                          cW1, cb1.reshape(1, -1), cW2, cb2.reshape(1, -1),
                          cW3, cb3.reshape(1, -1))
    return out
